# baseline (speedup 1.0000x reference)
import jax
import jax.numpy as jnp
from jax.experimental import pallas as pl

N = 100000
F = 128
BATCH = 100
MAXLEN = 1000
RB = 1000


def _copy_body(i_ref, o_ref):
    o_ref[...] = i_ref[...]


def kernel(attr, graph_id_attr, attr_len):
    out = pl.pallas_call(
        _copy_body,
        grid=(N // RB,),
        in_specs=[pl.BlockSpec((RB, F), lambda i: (i, 0))],
        out_specs=pl.BlockSpec((RB, F), lambda i: (i, 0)),
        out_shape=jax.ShapeDtypeStruct((N, F), jnp.float32),
    )(attr)
    return out.reshape(BATCH, MAXLEN, F)


# speedup vs baseline: 1.3656x; 1.3656x over previous
import functools

import jax
import jax.numpy as jnp
from jax import lax
from jax.experimental import pallas as pl
from jax.experimental.pallas import tpu as pltpu
from jax.experimental.pallas import tpu_sc as plsc

N = 100000
F = 128
BATCH = 100
MAXLEN = 1000
NC, NS, L = 2, 16, 16
NW = NC * NS
LEN_PAD = 112

BLK = 128
TEC_BLOCKS = 416
TEC_ROWS = TEC_BLOCKS * BLK
BASE_BLOCKS = TEC_BLOCKS // NW
EXTRA = TEC_BLOCKS % NW
MAXBLK = BASE_BLOCKS + 1
NBUF = 4
AHEAD = 2
NGROUPS = -(-MAXBLK // NBUF)

SCS_ROWS = N - TEC_ROWS
SCS_PER_CORE = SCS_ROWS // NC
CH = 1600
NFULL_CH = SCS_PER_CORE // CH
CH_TAIL = SCS_PER_CORE - NFULL_CH * CH
NCHUNK = NFULL_CH + 1
SBUF = 3
SAHEAD = 2

_vmesh = plsc.VectorSubcoreMesh(core_axis_name="c", subcore_axis_name="s",
                                num_cores=NC)
_smesh = plsc.ScalarSubcoreMesh(axis_name="c", num_cores=NC)


def _gather_lanes(v, idx):
    return lax.gather(
        v, idx[:, None],
        dimension_numbers=lax.GatherDimensionNumbers(
            offset_dims=(), collapsed_slice_dims=(0,), start_index_map=(0,)),
        slice_sizes=(1,),
        mode=lax.GatherScatterMode.PROMISE_IN_BOUNDS)


def _build_adj_regs(alen_v):
    iota = jnp.arange(L, dtype=jnp.int32)
    last = jnp.full((L,), L - 1, dtype=jnp.int32)
    carry = jnp.zeros((L,), jnp.int32)
    chunks = []
    for k in range(LEN_PAD // L):
        lv = alen_v[pl.ds(k * L, L)]
        s = lv
        for sh in (1, 2, 4, 8):
            shifted = _gather_lanes(s, jnp.maximum(iota - sh, 0))
            s = s + jnp.where(iota >= sh, shifted, 0)
        excl = s - lv + carry
        carry = carry + _gather_lanes(s, last)
        chunks.append((k * L + iota) * MAXLEN - excl)
    return chunks


def _lookup_adj(adj_chunks, g):
    st = jnp.zeros((L,), jnp.int32)
    for c, chunk in enumerate(adj_chunks):
        loc = g - (c * L)
        part = _gather_lanes(chunk, loc & (L - 1))
        st = jnp.where((loc >= 0) & (loc < L), part, st)
    return st


def _compute_indices(gbuf_r, adj_chunks, idx_r, base):
    for k in range(BLK // L):
        g = gbuf_r[pl.ds(k * L, L)]
        i_vec = base + (k * L) + jnp.arange(L, dtype=jnp.int32)
        idx_r[pl.ds(k * L, L)] = _lookup_adj(adj_chunks, g) + i_vec


def _tec_body(attr_hbm, gid_hbm, alen_hbm, out_hbm,
              alen_v, gbuf, idx_v, rows_v, spmem, *sems):
    del spmem
    load_sems = sems[:NBUF]
    scat_sems = sems[NBUF:2 * NBUF]
    wid = lax.axis_index("s") * NC + lax.axis_index("c")
    pltpu.sync_copy(alen_hbm, alen_v)
    adj_chunks = _build_adj_regs(alen_v)

    nblk = BASE_BLOCKS + jnp.where(wid < EXTRA, 1, 0)
    first = wid * BASE_BLOCKS + jnp.minimum(wid, EXTRA)

    def fire_load(t, b):
        base = (first + t) * BLK
        pltpu.async_copy(attr_hbm.at[pl.ds(base, BLK)], rows_v.at[b],
                         load_sems[b])
        pltpu.async_copy(gid_hbm.at[pl.ds(base, BLK)], gbuf.at[b],
                         load_sems[b])

    def wait_load(b):
        pltpu.make_async_copy(attr_hbm.at[pl.ds(0, BLK)], rows_v.at[b],
                              load_sems[b]).wait()
        pltpu.make_async_copy(gid_hbm.at[pl.ds(0, BLK)], gbuf.at[b],
                              load_sems[b]).wait()

    def wait_scat(b):
        pltpu.make_async_copy(rows_v.at[b], out_hbm.at[pl.ds(0, BLK)],
                              scat_sems[b]).wait()

    for b in range(AHEAD):
        fire_load(b, b)

    def group(gi, carry):
        for b0 in range(NBUF):
            t = gi * NBUF + b0
            b = b0

            @pl.when(t < nblk)
            def _process():
                wait_load(b)
                _compute_indices(gbuf.at[b], adj_chunks, idx_v.at[b],
                                 (first + t) * BLK)
                pltpu.async_copy(rows_v.at[b], out_hbm.at[idx_v.at[b]],
                                 scat_sems[b])

            t2 = t + AHEAD
            b2 = (b0 + AHEAD) % NBUF

            @pl.when(t2 < nblk)
            def _prefetch():
                @pl.when(t2 >= NBUF)
                def _drain():
                    wait_scat(b2)
                fire_load(t2, b2)

        return carry

    lax.fori_loop(0, NGROUPS, group, jnp.int32(0))

    for b in range(NBUF):
        wait_scat(b)


def _scs_body(attr_hbm, gid_hbm, alen_hbm, out_hbm,
              alen_v, gbuf, idx_v, rows_v, spmem, *sems):
    del alen_v, gbuf, idx_v, rows_v, gid_hbm, alen_hbm
    load_sems = sems[2 * NBUF:2 * NBUF + SBUF]
    store_sems = sems[2 * NBUF + SBUF:]
    cid = lax.axis_index("c")
    first = TEC_ROWS + cid * SCS_PER_CORE

    def chunk_rows(t):
        return CH if t < NFULL_CH else CH_TAIL

    def fire_load(t, b):
        n = chunk_rows(t)
        pltpu.async_copy(attr_hbm.at[pl.ds(first + t * CH, n)],
                         spmem.at[b, pl.ds(0, n)], load_sems[b])

    def wait_load(t, b):
        n = chunk_rows(t)
        pltpu.make_async_copy(attr_hbm.at[pl.ds(0, n)],
                              spmem.at[b, pl.ds(0, n)], load_sems[b]).wait()

    def fire_store(t, b):
        n = chunk_rows(t)
        pltpu.async_copy(spmem.at[b, pl.ds(0, n)],
                         out_hbm.at[pl.ds(first + t * CH, n)], store_sems[b])

    def wait_store(t, b):
        n = chunk_rows(t)
        pltpu.make_async_copy(spmem.at[b, pl.ds(0, n)],
                              out_hbm.at[pl.ds(0, n)], store_sems[b]).wait()

    for b in range(SAHEAD):
        fire_load(b, b)

    for t in range(NCHUNK):
        b = t % SBUF
        wait_load(t, b)
        fire_store(t, b)
        t2 = t + SAHEAD
        if t2 < NCHUNK:
            b2 = t2 % SBUF
            if t2 >= SBUF:
                wait_store(t2 - SBUF, b2)
            fire_load(t2, b2)

    for t in range(NCHUNK - SBUF, NCHUNK):
        wait_store(t, t % SBUF)


_scatter_kernel = pl.kernel(
    body=[_tec_body, _scs_body],
    mesh=[_vmesh, _smesh],
    out_type=jax.ShapeDtypeStruct((N, F), jnp.float32),
    scratch_types=(
        [
            (pltpu.VMEM @ _vmesh)((LEN_PAD,), jnp.int32),
            (pltpu.VMEM @ _vmesh)((NBUF, BLK), jnp.int32),
            (pltpu.VMEM @ _vmesh)((NBUF, BLK), jnp.int32),
            (pltpu.VMEM @ _vmesh)((NBUF, BLK, F), jnp.float32),
            pltpu.VMEM_SHARED((SBUF, CH, F), jnp.float32),
        ]
        + [pltpu.SemaphoreType.DMA @ _vmesh] * (2 * NBUF)
        + [pltpu.SemaphoreType.DMA @ _smesh] * (2 * SBUF)
    ),
)


def kernel(attr, graph_id_attr, attr_len):
    alen = jnp.pad(attr_len, (0, LEN_PAD - attr_len.shape[0]))
    out = _scatter_kernel(attr, graph_id_attr, alen)
    return out.reshape(BATCH, MAXLEN, F)


# speedup vs baseline: 1.3704x; 1.0035x over previous
import functools

import jax
import jax.numpy as jnp
from jax import lax
from jax.experimental import pallas as pl
from jax.experimental.pallas import tpu as pltpu
from jax.experimental.pallas import tpu_sc as plsc

N = 100000
F = 128
BATCH = 100
MAXLEN = 1000
NC, NS, L = 2, 16, 16
NW = NC * NS
LEN_PAD = 112

BLK = 128
TEC_BLOCKS = 448
TEC_ROWS = TEC_BLOCKS * BLK
BASE_BLOCKS = TEC_BLOCKS // NW
EXTRA = TEC_BLOCKS % NW
MAXBLK = BASE_BLOCKS + 1
NBUF = 4
AHEAD = 2
NGROUPS = -(-MAXBLK // NBUF)

SCS_ROWS = N - TEC_ROWS
SCS_PER_CORE = SCS_ROWS // NC
CH = 1600
NFULL_CH = SCS_PER_CORE // CH
CH_TAIL = SCS_PER_CORE - NFULL_CH * CH
NCHUNK = NFULL_CH + 1
SBUF = 3
SAHEAD = 2

_vmesh = plsc.VectorSubcoreMesh(core_axis_name="c", subcore_axis_name="s",
                                num_cores=NC)
_smesh = plsc.ScalarSubcoreMesh(axis_name="c", num_cores=NC)


def _gather_lanes(v, idx):
    return lax.gather(
        v, idx[:, None],
        dimension_numbers=lax.GatherDimensionNumbers(
            offset_dims=(), collapsed_slice_dims=(0,), start_index_map=(0,)),
        slice_sizes=(1,),
        mode=lax.GatherScatterMode.PROMISE_IN_BOUNDS)


def _build_adj_regs(alen_v):
    iota = jnp.arange(L, dtype=jnp.int32)
    last = jnp.full((L,), L - 1, dtype=jnp.int32)
    carry = jnp.zeros((L,), jnp.int32)
    chunks = []
    for k in range(LEN_PAD // L):
        lv = alen_v[pl.ds(k * L, L)]
        s = lv
        for sh in (1, 2, 4, 8):
            shifted = _gather_lanes(s, jnp.maximum(iota - sh, 0))
            s = s + jnp.where(iota >= sh, shifted, 0)
        excl = s - lv + carry
        carry = carry + _gather_lanes(s, last)
        chunks.append((k * L + iota) * MAXLEN - excl)
    return chunks


def _lookup_adj(adj_chunks, g):
    st = jnp.zeros((L,), jnp.int32)
    for c, chunk in enumerate(adj_chunks):
        loc = g - (c * L)
        part = _gather_lanes(chunk, loc & (L - 1))
        st = jnp.where((loc >= 0) & (loc < L), part, st)
    return st


def _compute_indices(gbuf_r, adj_chunks, idx_r, base):
    for k in range(BLK // L):
        g = gbuf_r[pl.ds(k * L, L)]
        i_vec = base + (k * L) + jnp.arange(L, dtype=jnp.int32)
        idx_r[pl.ds(k * L, L)] = _lookup_adj(adj_chunks, g) + i_vec


def _tec_body(attr_hbm, gid_hbm, alen_hbm, out_hbm,
              alen_v, gbuf, idx_v, rows_v, spmem, *sems):
    del spmem
    load_sems = sems[:NBUF]
    scat_sems = sems[NBUF:2 * NBUF]
    wid = lax.axis_index("s") * NC + lax.axis_index("c")
    pltpu.sync_copy(alen_hbm, alen_v)
    adj_chunks = _build_adj_regs(alen_v)

    nblk = BASE_BLOCKS + jnp.where(wid < EXTRA, 1, 0)
    first = wid * BASE_BLOCKS + jnp.minimum(wid, EXTRA)

    def fire_load(t, b):
        base = (first + t) * BLK
        pltpu.async_copy(attr_hbm.at[pl.ds(base, BLK)], rows_v.at[b],
                         load_sems[b])
        pltpu.async_copy(gid_hbm.at[pl.ds(base, BLK)], gbuf.at[b],
                         load_sems[b])

    def wait_load(b):
        pltpu.make_async_copy(attr_hbm.at[pl.ds(0, BLK)], rows_v.at[b],
                              load_sems[b]).wait()
        pltpu.make_async_copy(gid_hbm.at[pl.ds(0, BLK)], gbuf.at[b],
                              load_sems[b]).wait()

    def wait_scat(b):
        pltpu.make_async_copy(rows_v.at[b], out_hbm.at[pl.ds(0, BLK)],
                              scat_sems[b]).wait()

    for b in range(AHEAD):
        fire_load(b, b)

    def group(gi, carry):
        for b0 in range(NBUF):
            t = gi * NBUF + b0
            b = b0

            @pl.when(t < nblk)
            def _process():
                wait_load(b)
                _compute_indices(gbuf.at[b], adj_chunks, idx_v.at[b],
                                 (first + t) * BLK)
                pltpu.async_copy(rows_v.at[b], out_hbm.at[idx_v.at[b]],
                                 scat_sems[b])

            t2 = t + AHEAD
            b2 = (b0 + AHEAD) % NBUF

            @pl.when(t2 < nblk)
            def _prefetch():
                @pl.when(t2 >= NBUF)
                def _drain():
                    wait_scat(b2)
                fire_load(t2, b2)

        return carry

    lax.fori_loop(0, NGROUPS, group, jnp.int32(0))

    for b in range(NBUF):
        wait_scat(b)


def _scs_body(attr_hbm, gid_hbm, alen_hbm, out_hbm,
              alen_v, gbuf, idx_v, rows_v, spmem, *sems):
    del alen_v, gbuf, idx_v, rows_v, gid_hbm, alen_hbm
    load_sems = sems[2 * NBUF:2 * NBUF + SBUF]
    store_sems = sems[2 * NBUF + SBUF:]
    cid = lax.axis_index("c")
    first = TEC_ROWS + cid * SCS_PER_CORE

    def chunk_rows(t):
        return CH if t < NFULL_CH else CH_TAIL

    def fire_load(t, b):
        n = chunk_rows(t)
        pltpu.async_copy(attr_hbm.at[pl.ds(first + t * CH, n)],
                         spmem.at[b, pl.ds(0, n)], load_sems[b])

    def wait_load(t, b):
        n = chunk_rows(t)
        pltpu.make_async_copy(attr_hbm.at[pl.ds(0, n)],
                              spmem.at[b, pl.ds(0, n)], load_sems[b]).wait()

    def fire_store(t, b):
        n = chunk_rows(t)
        pltpu.async_copy(spmem.at[b, pl.ds(0, n)],
                         out_hbm.at[pl.ds(first + t * CH, n)], store_sems[b])

    def wait_store(t, b):
        n = chunk_rows(t)
        pltpu.make_async_copy(spmem.at[b, pl.ds(0, n)],
                              out_hbm.at[pl.ds(0, n)], store_sems[b]).wait()

    for b in range(SAHEAD):
        fire_load(b, b)

    for t in range(NCHUNK):
        b = t % SBUF
        wait_load(t, b)
        fire_store(t, b)
        t2 = t + SAHEAD
        if t2 < NCHUNK:
            b2 = t2 % SBUF
            if t2 >= SBUF:
                wait_store(t2 - SBUF, b2)
            fire_load(t2, b2)

    for t in range(NCHUNK - SBUF, NCHUNK):
        wait_store(t, t % SBUF)


_scatter_kernel = pl.kernel(
    body=[_tec_body, _scs_body],
    mesh=[_vmesh, _smesh],
    out_type=jax.ShapeDtypeStruct((N, F), jnp.float32),
    scratch_types=(
        [
            (pltpu.VMEM @ _vmesh)((LEN_PAD,), jnp.int32),
            (pltpu.VMEM @ _vmesh)((NBUF, BLK), jnp.int32),
            (pltpu.VMEM @ _vmesh)((NBUF, BLK), jnp.int32),
            (pltpu.VMEM @ _vmesh)((NBUF, BLK, F), jnp.float32),
            pltpu.VMEM_SHARED((SBUF, CH, F), jnp.float32),
        ]
        + [pltpu.SemaphoreType.DMA @ _vmesh] * (2 * NBUF)
        + [pltpu.SemaphoreType.DMA @ _smesh] * (2 * SBUF)
    ),
)


def kernel(attr, graph_id_attr, attr_len):
    alen = jnp.pad(attr_len, (0, LEN_PAD - attr_len.shape[0]))
    out = _scatter_kernel(attr, graph_id_attr, alen)
    return out.reshape(BATCH, MAXLEN, F)
